# baseline (device time: 97795 ns/iter reference)
import functools

import jax
import jax.numpy as jnp
from jax import lax
from jax.experimental import pallas as pl
from jax.experimental.pallas import tpu as pltpu

N_DEV = 8
N_SRC = 4
B, SQ, D = 2, 128, 512
HL, DH = 4, 64
SKV_SH = 128
QB = 64


def kernel(x, Wq, K_ext, V_ext, Wo):
    def body(
        x_ref, wq_ref, k_ref, v_ref, wo_ref, out_ref,
        kbuf, vbuf, pbuf, arbuf,
        ksend, vsend, krecv, vrecv, arsend, arrecv,
    ):
        my = lax.axis_index("i")
        is_even = (my % 2) == 0
        my_m = my // 2

        bar = pltpu.get_barrier_semaphore()
        for d in range(1, N_DEV):
            pl.semaphore_signal(
                bar, inc=1, device_id=((my + d) % N_DEV,),
                device_id_type=pl.DeviceIdType.MESH,
            )
        pl.semaphore_wait(bar, N_DEV - 1)

        @pl.when(is_even)
        def _():
            for d in range(1, N_DEV):
                dst = (my + d) % N_DEV
                for src_r, buf, ssem, rsem in (
                    (k_ref, kbuf, ksend, krecv),
                    (v_ref, vbuf, vsend, vrecv),
                ):
                    rdma = pltpu.make_async_remote_copy(
                        src_ref=src_r.at[:, :, pl.ds(dst * HL, HL), :],
                        dst_ref=buf.at[my_m],
                        send_sem=ssem.at[d],
                        recv_sem=rsem.at[my_m],
                        device_id=(dst,),
                        device_id_type=pl.DeviceIdType.MESH,
                    )
                    rdma.start()
            kbuf[my_m] = k_ref[:, :, pl.ds(my * HL, HL), :]
            vbuf[my_m] = v_ref[:, :, pl.ds(my * HL, HL), :]

        q = [
            jnp.dot(x_ref[b], wq_ref[:, :], preferred_element_type=jnp.float32)
            for b in range(B)
        ]

        for m in range(N_SRC):
            @pl.when(jnp.logical_not(jnp.logical_and(is_even, my_m == m)))
            def _():
                for buf, rsem in ((kbuf, krecv), (vbuf, vrecv)):
                    rd = pltpu.make_async_remote_copy(
                        src_ref=k_ref.at[:, :, pl.ds(0, HL), :],
                        dst_ref=buf.at[m],
                        send_sem=ksend.at[0],
                        recv_sem=rsem.at[m],
                        device_id=(0,),
                        device_id_type=pl.DeviceIdType.MESH,
                    )
                    rd.wait_recv()

        for b in range(B):
            for blk in range(2):
                acc = None
                for h in range(HL):
                    qb = q[b][blk * QB : (blk + 1) * QB, h * DH : (h + 1) * DH]
                    s_list = [
                        lax.dot_general(
                            qb,
                            kbuf[m, b, blk * QB : (blk + 1) * QB, h, :],
                            (((1,), (1,)), ((), ())),
                            preferred_element_type=jnp.float32,
                        )
                        * 0.125
                        for m in range(N_SRC)
                    ]
                    gmax = functools.reduce(
                        jnp.maximum,
                        [jnp.max(s, axis=-1, keepdims=True) for s in s_list],
                    )
                    w_list = [jnp.exp(s - gmax) for s in s_list]
                    den = functools.reduce(
                        jnp.add,
                        [jnp.sum(w, axis=-1, keepdims=True) for w in w_list],
                    )
                    ctx = functools.reduce(
                        jnp.add,
                        [
                            jnp.dot(
                                w_list[m],
                                vbuf[m, b, blk * QB : (blk + 1) * QB, h, :],
                                preferred_element_type=jnp.float32,
                            )
                            for m in range(N_SRC)
                        ],
                    ) / den
                    o = jnp.dot(
                        ctx,
                        wo_ref[h * DH : (h + 1) * DH, :],
                        preferred_element_type=jnp.float32,
                    )
                    acc = o if acc is None else acc + o
                pbuf[b, blk * QB : (blk + 1) * QB, :] = acc

        for d in range(1, N_DEV):
            dst = (my + d) % N_DEV
            ar = pltpu.make_async_remote_copy(
                src_ref=pbuf,
                dst_ref=arbuf.at[my],
                send_sem=arsend.at[d],
                recv_sem=arrecv.at[my],
                device_id=(dst,),
                device_id_type=pl.DeviceIdType.MESH,
            )
            ar.start()
        for d in range(1, N_DEV):
            src = (my + d) % N_DEV
            rd = pltpu.make_async_remote_copy(
                src_ref=pbuf,
                dst_ref=arbuf.at[src],
                send_sem=arsend.at[0],
                recv_sem=arrecv.at[src],
                device_id=(0,),
                device_id_type=pl.DeviceIdType.MESH,
            )
            rd.wait_recv()
        for b in range(B):
            tot = pbuf[b]
            for d in range(1, N_DEV):
                tot = tot + arbuf[(my + d) % N_DEV, b]
            out_ref[b] = tot

        @pl.when(is_even)
        def _():
            for d in range(1, N_DEV):
                for src_r, buf, ssem, rsem in (
                    (k_ref, kbuf, ksend, krecv),
                    (v_ref, vbuf, vsend, vrecv),
                ):
                    rd = pltpu.make_async_remote_copy(
                        src_ref=src_r.at[:, :, pl.ds(0, HL), :],
                        dst_ref=buf.at[0],
                        send_sem=ssem.at[d],
                        recv_sem=rsem.at[0],
                        device_id=(0,),
                        device_id_type=pl.DeviceIdType.MESH,
                    )
                    rd.wait_send()
        for d in range(1, N_DEV):
            rd = pltpu.make_async_remote_copy(
                src_ref=pbuf,
                dst_ref=arbuf.at[0],
                send_sem=arsend.at[d],
                recv_sem=arrecv.at[0],
                device_id=(0,),
                device_id_type=pl.DeviceIdType.MESH,
            )
            rd.wait_send()

        @functools.partial(pl.run_scoped, sem2=pltpu.SemaphoreType.REGULAR)
        def _(sem2):
            for d in range(1, N_DEV):
                pl.semaphore_signal(
                    sem2, inc=1, device_id=((my + d) % N_DEV,),
                    device_id_type=pl.DeviceIdType.MESH,
                )
            pl.semaphore_wait(sem2, N_DEV - 1)

    return pl.pallas_call(
        body,
        out_shape=jax.ShapeDtypeStruct((B, SQ, D), jnp.float32),
        in_specs=[pl.BlockSpec(memory_space=pltpu.VMEM)] * 5,
        out_specs=pl.BlockSpec(memory_space=pltpu.VMEM),
        scratch_shapes=[
            pltpu.VMEM((N_SRC, B, SKV_SH, HL, DH), jnp.float32),
            pltpu.VMEM((N_SRC, B, SKV_SH, HL, DH), jnp.float32),
            pltpu.VMEM((B, SQ, D), jnp.float32),
            pltpu.VMEM((N_DEV, B, SQ, D), jnp.float32),
            pltpu.SemaphoreType.DMA((N_DEV,)),
            pltpu.SemaphoreType.DMA((N_DEV,)),
            pltpu.SemaphoreType.DMA((N_SRC,)),
            pltpu.SemaphoreType.DMA((N_SRC,)),
            pltpu.SemaphoreType.DMA((N_DEV,)),
            pltpu.SemaphoreType.DMA((N_DEV,)),
        ],
        compiler_params=pltpu.CompilerParams(collective_id=0),
    )(x, Wq, K_ext, V_ext, Wo)


# device time: 89171 ns/iter; 1.0967x vs baseline; 1.0967x over previous
import functools

import jax
import jax.numpy as jnp
from jax import lax
from jax.experimental import pallas as pl
from jax.experimental.pallas import tpu as pltpu

N_DEV = 8
N_SRC = 4
B, SQ, D = 2, 128, 512
HL, DH = 4, 64
SKV_SH = 128
QB = 64


def kernel(x, Wq, K_ext, V_ext, Wo):
    def body(
        x_ref, wq_ref, k_ref, v_ref, wo_ref, out_ref,
        kbuf, vbuf, pbuf, rbuf1, rbuf2, rbuf3,
        ksend, vsend, krecv, vrecv, rssend, rsrecv, agsend, agrecv,
    ):
        my = lax.axis_index("i")
        is_even = (my % 2) == 0
        my_m = my // 2

        bar = pltpu.get_barrier_semaphore()
        for d in range(1, N_DEV):
            pl.semaphore_signal(
                bar, inc=1, device_id=((my + d) % N_DEV,),
                device_id_type=pl.DeviceIdType.MESH,
            )
        pl.semaphore_wait(bar, N_DEV - 1)

        @pl.when(is_even)
        def _():
            for d in range(1, N_DEV):
                dst = (my + d) % N_DEV
                for src_r, buf, ssem, rsem in (
                    (k_ref, kbuf, ksend, krecv),
                    (v_ref, vbuf, vsend, vrecv),
                ):
                    rdma = pltpu.make_async_remote_copy(
                        src_ref=src_r.at[:, :, pl.ds(dst * HL, HL), :],
                        dst_ref=buf.at[my_m],
                        send_sem=ssem.at[d],
                        recv_sem=rsem.at[my_m],
                        device_id=(dst,),
                        device_id_type=pl.DeviceIdType.MESH,
                    )
                    rdma.start()
            kbuf[my_m] = k_ref[:, :, pl.ds(my * HL, HL), :]
            vbuf[my_m] = v_ref[:, :, pl.ds(my * HL, HL), :]

        q = [
            jnp.dot(x_ref[b], wq_ref[:, :], preferred_element_type=jnp.float32)
            for b in range(B)
        ]

        for m in range(N_SRC):
            @pl.when(jnp.logical_not(jnp.logical_and(is_even, my_m == m)))
            def _():
                for buf, rsem in ((kbuf, krecv), (vbuf, vrecv)):
                    rd = pltpu.make_async_remote_copy(
                        src_ref=k_ref.at[:, :, pl.ds(0, HL), :],
                        dst_ref=buf.at[m],
                        send_sem=ksend.at[0],
                        recv_sem=rsem.at[m],
                        device_id=(0,),
                        device_id_type=pl.DeviceIdType.MESH,
                    )
                    rd.wait_recv()

        for b in range(B):
            for blk in range(2):
                acc = None
                for h in range(HL):
                    qb = q[b][blk * QB : (blk + 1) * QB, h * DH : (h + 1) * DH]
                    s_list = [
                        lax.dot_general(
                            qb,
                            kbuf[m, b, blk * QB : (blk + 1) * QB, h, :],
                            (((1,), (1,)), ((), ())),
                            preferred_element_type=jnp.float32,
                        )
                        * 0.125
                        for m in range(N_SRC)
                    ]
                    gmax = functools.reduce(
                        jnp.maximum,
                        [jnp.max(s, axis=-1, keepdims=True) for s in s_list],
                    )
                    w_list = [jnp.exp(s - gmax) for s in s_list]
                    den = functools.reduce(
                        jnp.add,
                        [jnp.sum(w, axis=-1, keepdims=True) for w in w_list],
                    )
                    ctx = functools.reduce(
                        jnp.add,
                        [
                            jnp.dot(
                                w_list[m],
                                vbuf[m, b, blk * QB : (blk + 1) * QB, h, :],
                                preferred_element_type=jnp.float32,
                            )
                            for m in range(N_SRC)
                        ],
                    ) / den
                    o = jnp.dot(
                        ctx,
                        wo_ref[h * DH : (h + 1) * DH, :],
                        preferred_element_type=jnp.float32,
                    )
                    acc = o if acc is None else acc + o
                pbuf[b, blk * QB : (blk + 1) * QB, :] = acc

        b0 = my % 2
        b1 = (my // 2) % 2
        b2 = my // 4
        o1 = b2 * 64
        o2 = o1 + b1 * 32
        o3 = o2 + b0 * 16

        rs_rounds = [
            (4, (1 - b2) * 64, o1, 64, rbuf1),
            (3, o1 + (1 - b1) * 32, o2, 32, rbuf2),
            (1, o2 + (1 - b0) * 16, o3, 16, rbuf3),
        ]
        for r, (mask, soff, koff, w, rbuf) in enumerate(rs_rounds):
            partner = jnp.bitwise_xor(my, mask)
            rdma = pltpu.make_async_remote_copy(
                src_ref=pbuf.at[:, pl.ds(soff, w), :],
                dst_ref=rbuf,
                send_sem=rssend.at[r],
                recv_sem=rsrecv.at[r],
                device_id=(partner,),
                device_id_type=pl.DeviceIdType.MESH,
            )
            rdma.start()
            rdma.wait()
            pbuf[:, pl.ds(koff, w), :] = pbuf[:, pl.ds(koff, w), :] + rbuf[:, :, :]

        out_ref[:, pl.ds(o3, 16), :] = pbuf[:, pl.ds(o3, 16), :]

        ag_rounds = [(1, o3, 16), (3, o2, 32), (4, o1, 64)]
        for r, (mask, off, w) in enumerate(ag_rounds):
            partner = jnp.bitwise_xor(my, mask)
            rdma = pltpu.make_async_remote_copy(
                src_ref=out_ref.at[:, pl.ds(off, w), :],
                dst_ref=out_ref.at[:, pl.ds(off, w), :],
                send_sem=agsend.at[r],
                recv_sem=agrecv.at[r],
                device_id=(partner,),
                device_id_type=pl.DeviceIdType.MESH,
            )
            rdma.start()
            rdma.wait()

        @pl.when(is_even)
        def _():
            for d in range(1, N_DEV):
                for src_r, buf, ssem, rsem in (
                    (k_ref, kbuf, ksend, krecv),
                    (v_ref, vbuf, vsend, vrecv),
                ):
                    rd = pltpu.make_async_remote_copy(
                        src_ref=src_r.at[:, :, pl.ds(0, HL), :],
                        dst_ref=buf.at[0],
                        send_sem=ssem.at[d],
                        recv_sem=rsem.at[0],
                        device_id=(0,),
                        device_id_type=pl.DeviceIdType.MESH,
                    )
                    rd.wait_send()

        @functools.partial(pl.run_scoped, sem2=pltpu.SemaphoreType.REGULAR)
        def _(sem2):
            for d in range(1, N_DEV):
                pl.semaphore_signal(
                    sem2, inc=1, device_id=((my + d) % N_DEV,),
                    device_id_type=pl.DeviceIdType.MESH,
                )
            pl.semaphore_wait(sem2, N_DEV - 1)

    return pl.pallas_call(
        body,
        out_shape=jax.ShapeDtypeStruct((B, SQ, D), jnp.float32),
        in_specs=[pl.BlockSpec(memory_space=pltpu.VMEM)] * 5,
        out_specs=pl.BlockSpec(memory_space=pltpu.VMEM),
        scratch_shapes=[
            pltpu.VMEM((N_SRC, B, SKV_SH, HL, DH), jnp.float32),
            pltpu.VMEM((N_SRC, B, SKV_SH, HL, DH), jnp.float32),
            pltpu.VMEM((B, SQ, D), jnp.float32),
            pltpu.VMEM((B, 64, D), jnp.float32),
            pltpu.VMEM((B, 32, D), jnp.float32),
            pltpu.VMEM((B, 16, D), jnp.float32),
            pltpu.SemaphoreType.DMA((N_DEV,)),
            pltpu.SemaphoreType.DMA((N_DEV,)),
            pltpu.SemaphoreType.DMA((N_SRC,)),
            pltpu.SemaphoreType.DMA((N_SRC,)),
            pltpu.SemaphoreType.DMA((3,)),
            pltpu.SemaphoreType.DMA((3,)),
            pltpu.SemaphoreType.DMA((3,)),
            pltpu.SemaphoreType.DMA((3,)),
        ],
        compiler_params=pltpu.CompilerParams(collective_id=0),
    )(x, Wq, K_ext, V_ext, Wo)
